# baseline (speedup 1.0000x reference)
import functools

import jax
import jax.numpy as jnp
from jax import lax
from jax.experimental import pallas as pl
from jax.experimental.pallas import tpu as pltpu
from jax.experimental.pallas import tpu_sc as plsc

VOCAB = 1_000_000
DIM = 64
SEQ = 50
NROW = 16384

NC = 2
NS = 16
NW = NC * NS

RPW = NROW // NW
R = 4
RSEQ = R * SEQ
K = RPW // R
NBUF = 8
OUTER = K // NBUF

_mesh = plsc.VectorSubcoreMesh(
    core_axis_name="c", subcore_axis_name="s", num_cores=NC, num_subcores=NS
)


@functools.partial(
    pl.kernel,
    out_type=jax.ShapeDtypeStruct((NROW, SEQ, DIM), jnp.float32),
    mesh=_mesh,
    scratch_types=[
        pltpu.VMEM((K, RSEQ), jnp.int32),
        pltpu.VMEM((NBUF, RSEQ, DIM), jnp.float32),
        [pltpu.SemaphoreType.DMA] * NBUF,
    ],
    compiler_params=pltpu.CompilerParams(use_tc_tiling_on_sc=False),
)
def _emb_lookup(idx_hbm, table_hbm, out_hbm, idx_v, rows_v, gsems):
    wid = lax.axis_index("s") * NC + lax.axis_index("c")
    base = wid * RPW
    pltpu.sync_copy(idx_hbm.at[wid], idx_v)

    def fire(kk, b):
        pltpu.async_copy(table_hbm.at[idx_v.at[kk]], rows_v.at[b], gsems[b])

    def drain(kk, b):
        pltpu.make_async_copy(table_hbm.at[idx_v.at[kk]], rows_v.at[b],
                              gsems[b]).wait()
        for r in range(R):
            pltpu.sync_copy(rows_v.at[b, pl.ds(r * SEQ, SEQ)],
                            out_hbm.at[base + kk * R + r])

    for b in range(NBUF):
        fire(b, b)

    @pl.loop(0, OUTER - 1)
    def _outer(o):
        for b in range(NBUF):
            kk = o * NBUF + b
            drain(kk, b)
            fire(kk + NBUF, b)

    for b in range(NBUF):
        drain((OUTER - 1) * NBUF + b, b)


def kernel(x, emb):
    return _emb_lookup(x.reshape(NW, K, RSEQ), emb)
